# baseline (speedup 1.0000x reference)
import dataclasses
import functools

import jax
import jax.numpy as jnp
from jax import lax
from jax.experimental import pallas as pl
from jax.experimental.pallas import tpu as pltpu
from jax.experimental.pallas import tpu_sc as plsc

_N = 1_000_000
_B = 2_000
_G = _B // 16
_GRID = _N // _B
_OBS_VAL = 1000.0


def _sc_indicator(x):
    mesh = plsc.VectorSubcoreMesh(core_axis_name="c", subcore_axis_name="s")
    cp = pltpu.CompilerParams()
    if "needs_layout_passes" in pltpu.CompilerParams.__dataclass_fields__:
        cp = dataclasses.replace(cp, needs_layout_passes=False)
    if "use_tc_tiling_on_sc" in pltpu.CompilerParams.__dataclass_fields__:
        cp = dataclasses.replace(cp, use_tc_tiling_on_sc=False)

    @functools.partial(
        pl.kernel,
        out_type=jax.ShapeDtypeStruct((_N, 1), jnp.float32),
        mesh=mesh,
        compiler_params=cp,
    )
    def sc_kernel(x_hbm, o_hbm):
        def body(x_vmem, o_vmem):
            lane = lax.iota(jnp.int32, 16)
            zeros = jnp.zeros((16,), jnp.int32)
            ones = jnp.ones((16,), jnp.int32)

            @pl.loop(0, _G)
            def _(g):
                rows = g * 16 + lane
                e = plsc.load_gather(x_vmem, [rows, zeros])
                o = plsc.load_gather(x_vmem, [rows, ones])
                m = (jnp.abs(e) <= 3.0) & (jnp.abs(o) <= 1.5)
                val = jnp.where(m, _OBS_VAL, 0.0).astype(jnp.float32)
                plsc.store_scatter(o_vmem, [rows, zeros], val)

        pltpu.emit_pipeline(
            body,
            grid=(_GRID,),
            in_specs=[pl.BlockSpec((_B, 2), lambda i: (i, 0))],
            out_specs=[pl.BlockSpec((_B, 1), lambda i: (i, 0))],
            core_axis_name=("c", "s"),
            dimension_semantics=(pltpu.PARALLEL,),
        )(x_hbm, o_hbm)

    return sc_kernel(x)


def kernel(x):
    return _sc_indicator(x)


# speedup vs baseline: 36.9595x; 36.9595x over previous
import dataclasses
import functools

import jax
import jax.numpy as jnp
from jax import lax
from jax.experimental import pallas as pl
from jax.experimental.pallas import tpu as pltpu
from jax.experimental.pallas import tpu_sc as plsc

_N = 1_000_000
_B = 4_000
_G = _B // 16
_GRID = _N // _B
_OBS_VAL = 1000.0


def _sc_indicator(xt):
    mesh = plsc.VectorSubcoreMesh(core_axis_name="c", subcore_axis_name="s")
    cp = pltpu.CompilerParams()
    if "needs_layout_passes" in pltpu.CompilerParams.__dataclass_fields__:
        cp = dataclasses.replace(cp, needs_layout_passes=False)
    if "use_tc_tiling_on_sc" in pltpu.CompilerParams.__dataclass_fields__:
        cp = dataclasses.replace(cp, use_tc_tiling_on_sc=False)

    @functools.partial(
        pl.kernel,
        out_type=jax.ShapeDtypeStruct((_N,), jnp.float32),
        mesh=mesh,
        compiler_params=cp,
    )
    def sc_kernel(x_hbm, o_hbm):
        def body(x_vmem, o_vmem):
            @pl.loop(0, _G)
            def _(g):
                sl = pl.ds(g * 16, 16)
                e = x_vmem[0, sl]
                o = x_vmem[1, sl]
                m = (jnp.abs(e) <= 3.0) & (jnp.abs(o) <= 1.5)
                o_vmem[sl] = jnp.where(m, _OBS_VAL, 0.0).astype(jnp.float32)

        pltpu.emit_pipeline(
            body,
            grid=(_GRID,),
            in_specs=[pl.BlockSpec((2, _B), lambda i: (0, i))],
            out_specs=[pl.BlockSpec((_B,), lambda i: (i,))],
            core_axis_name=("c", "s"),
            dimension_semantics=(pltpu.PARALLEL,),
        )(x_hbm, o_hbm)

    return sc_kernel(xt)


def kernel(x):
    out = _sc_indicator(x.T)
    return out.reshape(_N, 1)


# speedup vs baseline: 43.0963x; 1.1660x over previous
import dataclasses
import functools

import jax
import jax.numpy as jnp
from jax import lax
from jax.experimental import pallas as pl
from jax.experimental.pallas import tpu as pltpu
from jax.experimental.pallas import tpu_sc as plsc

_N = 1_000_000
_B = 8_000
_G = _B // 16
_GRID = _N // _B
_OBS_VAL = 1000.0


def _sc_indicator(xt):
    mesh = plsc.VectorSubcoreMesh(core_axis_name="c", subcore_axis_name="s")
    cp = pltpu.CompilerParams()
    if "needs_layout_passes" in pltpu.CompilerParams.__dataclass_fields__:
        cp = dataclasses.replace(cp, needs_layout_passes=False)
    if "use_tc_tiling_on_sc" in pltpu.CompilerParams.__dataclass_fields__:
        cp = dataclasses.replace(cp, use_tc_tiling_on_sc=False)

    @functools.partial(
        pl.kernel,
        out_type=jax.ShapeDtypeStruct((_N,), jnp.float32),
        mesh=mesh,
        compiler_params=cp,
    )
    def sc_kernel(x_hbm, o_hbm):
        def body(x_vmem, o_vmem):
            @plsc.parallel_loop(0, _G, 1, unroll=4)
            def _(g):
                sl = pl.ds(g * 16, 16)
                e = x_vmem[0, sl]
                o = x_vmem[1, sl]
                m = (jnp.abs(e) <= 3.0) & (jnp.abs(o) <= 1.5)
                o_vmem[sl] = jnp.where(m, _OBS_VAL, 0.0).astype(jnp.float32)

        pltpu.emit_pipeline(
            body,
            grid=(_GRID,),
            in_specs=[pl.BlockSpec((2, _B), lambda i: (0, i))],
            out_specs=[pl.BlockSpec((_B,), lambda i: (i,))],
            core_axis_name=("c", "s"),
            dimension_semantics=(pltpu.PARALLEL,),
        )(x_hbm, o_hbm)

    return sc_kernel(xt)


def kernel(x):
    out = _sc_indicator(x.T)
    return out.reshape(_N, 1)


# speedup vs baseline: 97.8575x; 2.2707x over previous
import jax
import jax.numpy as jnp
from jax.experimental import pallas as pl

_N = 1_000_000
_OBS_VAL = 1000.0


def _tc_indicator(xt):
    def body(x_ref, o_ref):
        e = x_ref[0, :]
        o = x_ref[1, :]
        m = (jnp.abs(e) <= 3.0) & (jnp.abs(o) <= 1.5)
        o_ref[...] = jnp.where(m, jnp.float32(_OBS_VAL), jnp.float32(0.0))

    return pl.pallas_call(
        body,
        out_shape=jax.ShapeDtypeStruct((_N,), jnp.float32),
    )(xt)


def kernel(x):
    out = _tc_indicator(x.T)
    return out.reshape(_N, 1)


# speedup vs baseline: 354.3388x; 3.6210x over previous
import jax
import jax.numpy as jnp
from jax.experimental import pallas as pl

_N = 1_000_000
_OBS_VAL = 1000.0


def _tc_indicator(xt):
    def body(x_ref, o_ref):
        e = x_ref[0, :]
        o = x_ref[1, :]
        m = (jnp.abs(e) <= 3.0) & (jnp.abs(o) <= 1.5)
        o_ref[0, :] = jnp.where(m, jnp.float32(_OBS_VAL), jnp.float32(0.0))

    return pl.pallas_call(
        body,
        out_shape=jax.ShapeDtypeStruct((1, _N), jnp.float32),
    )(xt)


def kernel(x):
    out = _tc_indicator(x.T)
    return out.reshape(_N, 1)
